# baseline (speedup 1.0000x reference)
import functools

import jax
import jax.numpy as jnp
from jax import lax
from jax.experimental import pallas as pl
from jax.experimental.pallas import tpu as pltpu
from jax.experimental.pallas import tpu_sc as plsc

N = 10000
E = 320000
D = 128

NC = 2
NS = 16
NW = NC * NS

B = 128
CPT = 80
E_PAD = NW * CPT * B
NCHUNK = E_PAD // B

N_PAD = 10240
DUMP_ROW = N_PAD - 1
ROWS_PER_TILE_SC = N_PAD // NS
G = 16
NBLK = CPT // G


def _sc_agg_body(with_deg, *refs):
    if with_deg:
        (x_hbm, edge_hbm, agg_out, deg_out,
         ib0, ib1, rows0, rows1, zblk, onesv, agg_sh, deg_sh,
         semI0, semI1, semR0, semR1) = refs
    else:
        (x_hbm, edge_hbm, agg_out,
         ib0, ib1, rows0, rows1, zblk, onesv, agg_sh, deg_sh,
         semI0, semI1, semR0, semR1) = refs
    ibufs = (ib0, ib1)
    isems = (semI0, semI1)

    cid = lax.axis_index("c")
    sid = lax.axis_index("s")
    wid = sid * NC + cid

    def fire_idx(k):
        pltpu.async_copy(edge_hbm.at[:, pl.ds(wid * CPT + k * G, G), :],
                         ibufs[k % 2], isems[k % 2])

    def wait_idx(k):
        pltpu.make_async_copy(
            edge_hbm.at[:, pl.ds(wid * CPT + k * G, G), :],
            ibufs[k % 2], isems[k % 2]).wait()

    fire_idx(0)
    fire_idx(1)

    zero16 = jnp.zeros((16,), jnp.float32)
    for r in range(16):
        for c8 in range(D // 16):
            zblk[r, pl.ds(c8 * 16, 16)] = zero16
    one16 = jnp.ones((16,), jnp.float32)
    for c8 in range(B // 16):
        onesv[pl.ds(c8 * 16, 16)] = one16

    zbase = sid * ROWS_PER_TILE_SC

    def zloop(i, carry):
        pltpu.sync_copy(zblk, agg_sh.at[pl.ds(zbase + i * 16, 16)])
        return carry

    lax.fori_loop(0, ROWS_PER_TILE_SC // 16, zloop, 0)
    if with_deg:
        def zdloop(i, carry):
            pltpu.sync_copy(zblk.at[0], deg_sh.at[pl.ds(zbase + i * D, D)])
            return carry

        lax.fori_loop(0, ROWS_PER_TILE_SC // D, zdloop, 0)
    plsc.subcore_barrier()

    def fire(ib, j, rowsv, sem):
        pltpu.async_copy(x_hbm.at[ib.at[0, j]], rowsv, sem)

    def drain(ib, j, rowsv, sem):
        pltpu.make_async_copy(x_hbm.at[ib.at[0, j]], rowsv, sem).wait()
        pltpu.sync_copy(rowsv, agg_sh.at[ib.at[1, j]], add=True)
        if with_deg:
            pltpu.sync_copy(onesv, deg_sh.at[ib.at[1, j]], add=True)

    for k in range(NBLK):
        ib = ibufs[k % 2]
        wait_idx(k)
        fire(ib, 0, rows0, semR0)

        def inner(m, carry, ib=ib):
            fire(ib, 2 * m + 1, rows1, semR1)
            drain(ib, 2 * m, rows0, semR0)
            fire(ib, 2 * m + 2, rows0, semR0)
            drain(ib, 2 * m + 1, rows1, semR1)
            return carry

        lax.fori_loop(0, G // 2 - 1, inner, 0)
        fire(ib, G - 1, rows1, semR1)
        drain(ib, G - 2, rows0, semR0)
        drain(ib, G - 1, rows1, semR1)
        if k + 2 < NBLK:
            fire_idx(k + 2)
    plsc.subcore_barrier()

    pltpu.sync_copy(agg_sh.at[pl.ds(zbase, ROWS_PER_TILE_SC)],
                    agg_out.at[cid, pl.ds(zbase, ROWS_PER_TILE_SC)])
    if with_deg:
        pltpu.sync_copy(deg_sh.at[pl.ds(zbase, ROWS_PER_TILE_SC)],
                        deg_out.at[cid, pl.ds(zbase, ROWS_PER_TILE_SC)])


@functools.lru_cache(maxsize=None)
def _make_sc_agg(with_deg):
    mesh = plsc.VectorSubcoreMesh(core_axis_name="c", subcore_axis_name="s")
    out_type = [jax.ShapeDtypeStruct((NC, N_PAD, D), jnp.float32)]
    if with_deg:
        out_type.append(jax.ShapeDtypeStruct((NC, N_PAD), jnp.float32))
    scratch = [
        pltpu.VMEM((2, G, B), jnp.int32),
        pltpu.VMEM((2, G, B), jnp.int32),
        pltpu.VMEM((B, D), jnp.float32),
        pltpu.VMEM((B, D), jnp.float32),
        pltpu.VMEM((16, D), jnp.float32),
        pltpu.VMEM((B,), jnp.float32),
        pltpu.VMEM_SHARED((N_PAD, D), jnp.float32),
        pltpu.VMEM_SHARED((N_PAD,), jnp.float32),
        pltpu.SemaphoreType.DMA,
        pltpu.SemaphoreType.DMA,
        pltpu.SemaphoreType.DMA,
        pltpu.SemaphoreType.DMA,
    ]
    return pl.kernel(
        functools.partial(_sc_agg_body, with_deg),
        mesh=mesh,
        out_type=out_type if with_deg else out_type[0],
        scratch_types=scratch,
    )


R_BLK = 1024


def _dense_body(relu, agg_ref, deg_ref, x_ref, wl_ref, b_ref, wr_ref, o_ref):
    agg = agg_ref[0] + agg_ref[1]
    deg = deg_ref[0] + deg_ref[1]
    inv = 1.0 / jnp.maximum(deg, 1.0)
    mean = (agg.reshape(R_BLK // 128, 128, D) * inv[:, :, None]).reshape(R_BLK, D)
    out = jnp.dot(mean, wl_ref[...], preferred_element_type=jnp.float32)
    out = out + jnp.dot(x_ref[...], wr_ref[...], preferred_element_type=jnp.float32)
    out = out + b_ref[...]
    if relu:
        out = jnp.maximum(out, 0.0)
    o_ref[...] = out


def _dense(aggp, degp3, x_pad, wlT, b2, wrT, relu):
    grid = (N_PAD // R_BLK,)
    return pl.pallas_call(
        functools.partial(_dense_body, relu),
        grid=grid,
        in_specs=[
            pl.BlockSpec((NC, R_BLK, D), lambda i: (0, i, 0)),
            pl.BlockSpec((NC, R_BLK // 128, 128), lambda i: (0, i, 0)),
            pl.BlockSpec((R_BLK, D), lambda i: (i, 0)),
            pl.BlockSpec((D, D), lambda i: (0, 0)),
            pl.BlockSpec((1, D), lambda i: (0, 0)),
            pl.BlockSpec((D, D), lambda i: (0, 0)),
        ],
        out_specs=pl.BlockSpec((R_BLK, D), lambda i: (i, 0)),
        out_shape=jax.ShapeDtypeStruct((N_PAD, D), jnp.float32),
    )(aggp, degp3, x_pad, wlT, b2, wrT)


def kernel(x, edge_index, W_l1, b_l1, W_r1, W_l2, b_l2, W_r2):
    x_pad = jnp.pad(x, ((0, N_PAD - N), (0, 0)))
    pad_col = jnp.array([[0], [DUMP_ROW]], dtype=jnp.int32)
    edge3 = jnp.concatenate(
        [edge_index, jnp.broadcast_to(pad_col, (2, E_PAD - E))], axis=1
    ).reshape(2, NCHUNK, B)

    aggp1, degp = _make_sc_agg(True)(x_pad, edge3)
    degp3 = degp.reshape(NC, N_PAD // 128, 128)
    h = _dense(aggp1, degp3, x_pad, W_l1.T, b_l1.reshape(1, D), W_r1.T, True)

    aggp2 = _make_sc_agg(False)(h, edge3)
    out = _dense(aggp2, degp3, h, W_l2.T, b_l2.reshape(1, D), W_r2.T, False)
    return out[:N]


# speedup vs baseline: 3.4388x; 3.4388x over previous
import functools

import jax
import jax.numpy as jnp
from jax import lax
from jax.experimental import pallas as pl
from jax.experimental.pallas import tpu as pltpu
from jax.experimental.pallas import tpu_sc as plsc

N = 10000
E = 320000
D = 128

NC = 2
NS = 16
NW = NC * NS

B = 128
CPT = 80
E_PAD = NW * CPT * B
NCHUNK = E_PAD // B

N_PAD = 10240
DUMP_ROW = N_PAD - 1
ROWS_PER_TILE_SC = N_PAD // NS
G = 16
NBLK = CPT // G


def _sc_agg_body(with_deg, *refs):
    if with_deg:
        (x_hbm, edge_hbm, agg_out, deg_out,
         ib0, ib1, rows0, rows1, zblk, onesv, agg_sh, deg_sh,
         semI0, semI1, semR0, semR1) = refs
    else:
        (x_hbm, edge_hbm, agg_out,
         ib0, ib1, rows0, rows1, zblk, onesv, agg_sh, deg_sh,
         semI0, semI1, semR0, semR1) = refs
    ibufs = (ib0, ib1)
    isems = (semI0, semI1)

    cid = lax.axis_index("c")
    sid = lax.axis_index("s")
    wid = sid * NC + cid

    def fire_idx(k):
        pltpu.async_copy(edge_hbm.at[:, pl.ds(wid * CPT + k * G, G), :],
                         ibufs[k % 2], isems[k % 2])

    def wait_idx(k):
        pltpu.make_async_copy(
            edge_hbm.at[:, pl.ds(wid * CPT + k * G, G), :],
            ibufs[k % 2], isems[k % 2]).wait()

    fire_idx(0)
    fire_idx(1)

    zero16 = jnp.zeros((16,), jnp.float32)
    for r in range(16):
        for c8 in range(D // 16):
            zblk[r, pl.ds(c8 * 16, 16)] = zero16
    one16 = jnp.ones((16,), jnp.float32)
    for c8 in range(B // 16):
        onesv[pl.ds(c8 * 16, 16)] = one16

    zbase = sid * ROWS_PER_TILE_SC

    def zloop(i, carry):
        pltpu.sync_copy(zblk, agg_sh.at[pl.ds(zbase + i * 16, 16)])
        return carry

    lax.fori_loop(0, ROWS_PER_TILE_SC // 16, zloop, 0)
    if with_deg:
        def zdloop(i, carry):
            pltpu.sync_copy(zblk.at[0], deg_sh.at[pl.ds(zbase + i * D, D)])
            return carry

        lax.fori_loop(0, ROWS_PER_TILE_SC // D, zdloop, 0)
    plsc.subcore_barrier()

    def fire(ib, j, rowsv, sem):
        pltpu.async_copy(x_hbm.at[ib.at[0, j]], rowsv, sem)

    def drain(ib, j, rowsv, sem):
        pltpu.make_async_copy(x_hbm.at[ib.at[0, j]], rowsv, sem).wait()
        pltpu.sync_copy(rowsv, agg_sh.at[ib.at[1, j]], add=True)
        if with_deg:
            pltpu.sync_copy(onesv, deg_sh.at[ib.at[1, j]], add=True)

    for k in range(NBLK):
        ib = ibufs[k % 2]
        wait_idx(k)
        fire(ib, 0, rows0, semR0)

        def inner(m, carry, ib=ib):
            fire(ib, 2 * m + 1, rows1, semR1)
            drain(ib, 2 * m, rows0, semR0)
            fire(ib, 2 * m + 2, rows0, semR0)
            drain(ib, 2 * m + 1, rows1, semR1)
            return carry

        lax.fori_loop(0, G // 2 - 1, inner, 0)
        fire(ib, G - 1, rows1, semR1)
        drain(ib, G - 2, rows0, semR0)
        drain(ib, G - 1, rows1, semR1)
        if k + 2 < NBLK:
            fire_idx(k + 2)
    plsc.subcore_barrier()

    pltpu.sync_copy(agg_sh.at[pl.ds(zbase, ROWS_PER_TILE_SC)],
                    agg_out.at[cid, pl.ds(zbase, ROWS_PER_TILE_SC)])
    if with_deg:
        pltpu.sync_copy(deg_sh.at[pl.ds(zbase, ROWS_PER_TILE_SC)],
                        deg_out.at[cid, pl.ds(zbase, ROWS_PER_TILE_SC)])


@functools.lru_cache(maxsize=None)
def _make_sc_agg(with_deg):
    mesh = plsc.VectorSubcoreMesh(core_axis_name="c", subcore_axis_name="s")
    out_type = [jax.ShapeDtypeStruct((NC, N_PAD, D), jnp.float32)]
    if with_deg:
        out_type.append(jax.ShapeDtypeStruct((NC, N_PAD), jnp.float32))
    scratch = [
        pltpu.VMEM((2, G, B), jnp.int32),
        pltpu.VMEM((2, G, B), jnp.int32),
        pltpu.VMEM((B, D), jnp.float32),
        pltpu.VMEM((B, D), jnp.float32),
        pltpu.VMEM((16, D), jnp.float32),
        pltpu.VMEM((B,), jnp.float32),
        pltpu.VMEM_SHARED((N_PAD, D), jnp.float32),
        pltpu.VMEM_SHARED((N_PAD,), jnp.float32),
        pltpu.SemaphoreType.DMA,
        pltpu.SemaphoreType.DMA,
        pltpu.SemaphoreType.DMA,
        pltpu.SemaphoreType.DMA,
    ]
    return pl.kernel(
        functools.partial(_sc_agg_body, with_deg),
        mesh=mesh,
        out_type=out_type if with_deg else out_type[0],
        scratch_types=scratch,
    )


R_BLK = 1024


def _dense_body(relu, agg_ref, deg_ref, x_ref, wl_ref, b_ref, wr_ref, o_ref):
    agg = agg_ref[0] + agg_ref[1]
    deg = deg_ref[0] + deg_ref[1]
    inv = 1.0 / jnp.maximum(deg, 1.0)
    mean = (agg.reshape(R_BLK // 128, 128, D) * inv[:, :, None]).reshape(R_BLK, D)
    out = jnp.dot(mean, wl_ref[...], preferred_element_type=jnp.float32)
    out = out + jnp.dot(x_ref[...], wr_ref[...], preferred_element_type=jnp.float32)
    out = out + b_ref[...]
    if relu:
        out = jnp.maximum(out, 0.0)
    o_ref[...] = out


def _dense(aggp, degp3, x_pad, wlT, b2, wrT, relu):
    grid = (N_PAD // R_BLK,)
    return pl.pallas_call(
        functools.partial(_dense_body, relu),
        grid=grid,
        in_specs=[
            pl.BlockSpec((NC, R_BLK, D), lambda i: (0, i, 0)),
            pl.BlockSpec((NC, R_BLK // 128, 128), lambda i: (0, i, 0)),
            pl.BlockSpec((R_BLK, D), lambda i: (i, 0)),
            pl.BlockSpec((D, D), lambda i: (0, 0)),
            pl.BlockSpec((1, D), lambda i: (0, 0)),
            pl.BlockSpec((D, D), lambda i: (0, 0)),
        ],
        out_specs=pl.BlockSpec((R_BLK, D), lambda i: (i, 0)),
        out_shape=jax.ShapeDtypeStruct((N_PAD, D), jnp.float32),
    )(aggp, degp3, x_pad, wlT, b2, wrT)


def kernel(x, edge_index, W_l1, b_l1, W_r1, W_l2, b_l2, W_r2):
    x_pad = jnp.pad(x, ((0, N_PAD - N), (0, 0)))
    npad_e = E_PAD - E
    pad_iota = lax.iota(jnp.int32, npad_e)
    pad_edges = jnp.stack([pad_iota % N, N + pad_iota % (N_PAD - N)])
    edge3 = jnp.concatenate([edge_index, pad_edges], axis=1
                            ).reshape(2, NCHUNK, B)

    aggp1, degp = _make_sc_agg(True)(x_pad, edge3)
    degp3 = degp.reshape(NC, N_PAD // 128, 128)
    h = _dense(aggp1, degp3, x_pad, W_l1.T, b_l1.reshape(1, D), W_r1.T, True)

    aggp2 = _make_sc_agg(False)(h, edge3)
    out = _dense(aggp2, degp3, h, W_l2.T, b_l2.reshape(1, D), W_r2.T, False)
    return out[:N]


# speedup vs baseline: 3.4504x; 1.0034x over previous
import functools

import jax
import jax.numpy as jnp
from jax import lax
from jax.experimental import pallas as pl
from jax.experimental.pallas import tpu as pltpu
from jax.experimental.pallas import tpu_sc as plsc

N = 10000
E = 320000
D = 128

NC = 2
NS = 16
NW = NC * NS

B = 128
CPT = 80
E_PAD = NW * CPT * B
NCHUNK = E_PAD // B

N_PAD = 10240
DUMP_ROW = N_PAD - 1
ROWS_PER_TILE_SC = N_PAD // NS
G = 16
NBLK = CPT // G


def _sc_agg_body(with_deg, *refs):
    if with_deg:
        (x_hbm, edge_hbm, agg_out, deg_out,
         ib0, ib1, rows0, rows1, zblk, onesv, agg_sh, deg_sh,
         semI0, semI1, semR0, semR1) = refs
    else:
        (x_hbm, edge_hbm, agg_out,
         ib0, ib1, rows0, rows1, zblk, onesv, agg_sh, deg_sh,
         semI0, semI1, semR0, semR1) = refs
    ibufs = (ib0, ib1)
    isems = (semI0, semI1)

    cid = lax.axis_index("c")
    sid = lax.axis_index("s")
    wid = sid * NC + cid

    def fire_idx(k):
        pltpu.async_copy(edge_hbm.at[:, pl.ds(wid * CPT + k * G, G), :],
                         ibufs[k % 2], isems[k % 2])

    def wait_idx(k):
        pltpu.make_async_copy(
            edge_hbm.at[:, pl.ds(wid * CPT + k * G, G), :],
            ibufs[k % 2], isems[k % 2]).wait()

    fire_idx(0)
    fire_idx(1)

    zero16 = jnp.zeros((16,), jnp.float32)
    for r in range(16):
        for c8 in range(D // 16):
            zblk[r, pl.ds(c8 * 16, 16)] = zero16
    one16 = jnp.ones((16,), jnp.float32)
    for c8 in range(B // 16):
        onesv[pl.ds(c8 * 16, 16)] = one16

    zbase = sid * ROWS_PER_TILE_SC
    ZROWS = 16

    def zloop(i, carry):
        pltpu.async_copy(zblk, agg_sh.at[pl.ds(zbase + i * ZROWS, ZROWS)],
                         semR0)
        return carry

    nz = ROWS_PER_TILE_SC // ZROWS
    lax.fori_loop(0, nz, zloop, 0)
    if with_deg:
        def zdloop(i, carry):
            pltpu.async_copy(zblk.at[0], deg_sh.at[pl.ds(zbase + i * D, D)],
                             semR1)
            return carry

        lax.fori_loop(0, ROWS_PER_TILE_SC // D, zdloop, 0)

    def zdrain(i, carry):
        pltpu.make_async_copy(
            zblk, agg_sh.at[pl.ds(zbase + i * ZROWS, ZROWS)], semR0).wait()
        return carry

    lax.fori_loop(0, nz, zdrain, 0)
    if with_deg:
        def zddrain(i, carry):
            pltpu.make_async_copy(
                zblk.at[0], deg_sh.at[pl.ds(zbase + i * D, D)], semR1).wait()
            return carry

        lax.fori_loop(0, ROWS_PER_TILE_SC // D, zddrain, 0)
    plsc.subcore_barrier()

    def fire(ib, j, rowsv, sem):
        pltpu.async_copy(x_hbm.at[ib.at[0, j]], rowsv, sem)

    def drain(ib, j, rowsv, sem):
        pltpu.make_async_copy(x_hbm.at[ib.at[0, j]], rowsv, sem).wait()
        pltpu.sync_copy(rowsv, agg_sh.at[ib.at[1, j]], add=True)
        if with_deg:
            pltpu.sync_copy(onesv, deg_sh.at[ib.at[1, j]], add=True)
        if with_deg:
            pltpu.sync_copy(onesv, deg_sh.at[ib.at[1, j]], add=True)

    for k in range(NBLK):
        ib = ibufs[k % 2]
        wait_idx(k)
        fire(ib, 0, rows0, semR0)

        def inner(m, carry, ib=ib):
            fire(ib, 2 * m + 1, rows1, semR1)
            drain(ib, 2 * m, rows0, semR0)
            fire(ib, 2 * m + 2, rows0, semR0)
            drain(ib, 2 * m + 1, rows1, semR1)
            return carry

        lax.fori_loop(0, G // 2 - 1, inner, 0)
        fire(ib, G - 1, rows1, semR1)
        drain(ib, G - 2, rows0, semR0)
        drain(ib, G - 1, rows1, semR1)
        if k + 2 < NBLK:
            fire_idx(k + 2)
    plsc.subcore_barrier()

    pltpu.sync_copy(agg_sh.at[pl.ds(zbase, ROWS_PER_TILE_SC)],
                    agg_out.at[cid, pl.ds(zbase, ROWS_PER_TILE_SC)])
    if with_deg:
        pltpu.sync_copy(deg_sh.at[pl.ds(zbase, ROWS_PER_TILE_SC)],
                        deg_out.at[cid, pl.ds(zbase, ROWS_PER_TILE_SC)])


@functools.lru_cache(maxsize=None)
def _make_sc_agg(with_deg):
    mesh = plsc.VectorSubcoreMesh(core_axis_name="c", subcore_axis_name="s")
    out_type = [jax.ShapeDtypeStruct((NC, N_PAD, D), jnp.float32)]
    if with_deg:
        out_type.append(jax.ShapeDtypeStruct((NC, N_PAD), jnp.float32))
    scratch = [
        pltpu.VMEM((2, G, B), jnp.int32),
        pltpu.VMEM((2, G, B), jnp.int32),
        pltpu.VMEM((B, D), jnp.float32),
        pltpu.VMEM((B, D), jnp.float32),
        pltpu.VMEM((16, D), jnp.float32),
        pltpu.VMEM((B,), jnp.float32),
        pltpu.VMEM_SHARED((N_PAD, D), jnp.float32),
        pltpu.VMEM_SHARED((N_PAD,), jnp.float32),
        pltpu.SemaphoreType.DMA,
        pltpu.SemaphoreType.DMA,
        pltpu.SemaphoreType.DMA,
        pltpu.SemaphoreType.DMA,
    ]
    return pl.kernel(
        functools.partial(_sc_agg_body, with_deg),
        mesh=mesh,
        out_type=out_type if with_deg else out_type[0],
        scratch_types=scratch,
    )


R_BLK = 1000


def _dense_body(relu, agg_ref, deg_ref, x_ref, wl_ref, b_ref, wr_ref, o_ref):
    agg = agg_ref[0] + agg_ref[1]
    deg = deg_ref[0] + deg_ref[1]
    inv = 1.0 / jnp.maximum(deg, 1.0)
    mean = agg * inv
    out = jnp.dot(mean, wl_ref[...], preferred_element_type=jnp.float32)
    out = out + jnp.dot(x_ref[...], wr_ref[...], preferred_element_type=jnp.float32)
    out = out + b_ref[...]
    if relu:
        out = jnp.maximum(out, 0.0)
    o_ref[...] = out


def _dense(aggp, degp3, x_in, wlT, b2, wrT, relu):
    grid = (N // R_BLK,)
    return pl.pallas_call(
        functools.partial(_dense_body, relu),
        grid=grid,
        in_specs=[
            pl.BlockSpec((NC, R_BLK, D), lambda i: (0, i, 0)),
            pl.BlockSpec((NC, R_BLK, 1), lambda i: (0, i, 0)),
            pl.BlockSpec((R_BLK, D), lambda i: (i, 0)),
            pl.BlockSpec((D, D), lambda i: (0, 0)),
            pl.BlockSpec((1, D), lambda i: (0, 0)),
            pl.BlockSpec((D, D), lambda i: (0, 0)),
        ],
        out_specs=pl.BlockSpec((R_BLK, D), lambda i: (i, 0)),
        out_shape=jax.ShapeDtypeStruct((N, D), jnp.float32),
    )(aggp, degp3, x_in, wlT, b2, wrT)


def kernel(x, edge_index, W_l1, b_l1, W_r1, W_l2, b_l2, W_r2):
    npad_e = E_PAD - E
    pad_iota = lax.iota(jnp.int32, npad_e)
    pad_edges = jnp.stack([pad_iota % N, N + pad_iota % (N_PAD - N)])
    edge3 = jnp.concatenate([edge_index, pad_edges], axis=1
                            ).reshape(2, NCHUNK, B)

    aggp1, degp = _make_sc_agg(True)(x, edge3)
    degp3 = degp.reshape(NC, N_PAD, 1)
    h = _dense(aggp1, degp3, x, W_l1.T, b_l1.reshape(1, D), W_r1.T, True)

    aggp2 = _make_sc_agg(False)(h, edge3)
    return _dense(aggp2, degp3, h, W_l2.T, b_l2.reshape(1, D), W_r2.T, False)
